# Initial kernel scaffold; baseline (speedup 1.0000x reference)
import jax
import jax.numpy as jnp
from jax.experimental import pallas as pl


def kernel(x, W1, b1, g1, be1, Wl0, bl0, Wc, bc, gc, bec, Wl1, bl1, edge_index, batch):
    raise NotImplementedError("write your pallas kernel here")


# speedup vs baseline: 4.7446x; 4.7446x over previous
import functools

import jax
import jax.numpy as jnp
from jax import lax
from jax.experimental import pallas as pl
from jax.experimental.pallas import tpu as pltpu
from jax.experimental.pallas import tpu_sc as plsc

N = 10000
E = 320000
D = 128
H = 128
T = 2
G = 8

NC = 2
NS = 16
NW = NC * NS
EPW = E // NW
K = 80
NCHUNK = EPW // K
NPAD = 10112
RPT = NPAD // NS


def _stage1_body(x_ref, w1_ref, b1_ref, g1_ref, be1_ref, feat_ref):
    h = lax.dot_general(x_ref[...], w1_ref[...], (((1,), (1,)), ((), ())),
                        precision=lax.Precision.HIGHEST)
    h = h + b1_ref[...]
    mu = jnp.mean(h, axis=0, keepdims=True)
    var = jnp.mean((h - mu) ** 2, axis=0, keepdims=True)
    hn = (h - mu) * lax.rsqrt(var + 1e-5)
    feat_ref[...] = jnp.maximum(g1_ref[...] * hn + be1_ref[...], 0.0)


_stage1 = pl.pallas_call(
    _stage1_body,
    out_shape=jax.ShapeDtypeStruct((N, D), jnp.float32),
)


def _sc_agg_body(src_hbm, dst_hbm, feat_hbm, zeros_hbm, out_hbm,
                 sidx, didx, rows, accum, sem):
    c = lax.axis_index("c")
    s = lax.axis_index("s")
    wid = c * NS + s

    pltpu.sync_copy(zeros_hbm.at[pl.ds(s * RPT, RPT)], accum.at[pl.ds(s * RPT, RPT)])
    plsc.subcore_barrier()

    def chunk(i, carry):
        base = wid * EPW + i * K
        pltpu.sync_copy(src_hbm.at[pl.ds(base, K)], sidx)
        pltpu.async_copy(feat_hbm.at[sidx], rows, sem).wait()
        pltpu.sync_copy(dst_hbm.at[pl.ds(base, K)], didx)
        pltpu.sync_copy(rows, accum.at[didx], add=True)
        return carry

    lax.fori_loop(0, NCHUNK, chunk, 0)
    plsc.subcore_barrier()

    pltpu.sync_copy(accum.at[pl.ds(s * RPT, RPT)],
                    out_hbm.at[pl.ds(c * NPAD + s * RPT, RPT)])


_sc_agg = functools.partial(
    pl.kernel,
    out_type=jax.ShapeDtypeStruct((2 * NPAD, D), jnp.float32),
    mesh=plsc.VectorSubcoreMesh(core_axis_name="c", subcore_axis_name="s",
                                num_cores=NC, num_subcores=NS),
    scratch_types=[
        pltpu.VMEM((K,), jnp.int32),
        pltpu.VMEM((K,), jnp.int32),
        pltpu.VMEM((K, D), jnp.float32),
        pltpu.VMEM_SHARED((NPAD, D), jnp.float32),
        pltpu.SemaphoreType.DMA,
    ],
)(_sc_agg_body)


def _stage2a_body(feat_ref, agg_ref, wc_ref, bc_ref, gc_ref, bec_ref,
                  wl1_ref, bl1_ref, np1_ref):
    h = feat_ref[...] + agg_ref[0:N, :] + agg_ref[NPAD:NPAD + N, :]
    h = lax.dot_general(h, wc_ref[...], (((1,), (1,)), ((), ())),
                        precision=lax.Precision.HIGHEST)
    h = h + bc_ref[...]
    mu = jnp.mean(h, axis=0, keepdims=True)
    var = jnp.mean((h - mu) ** 2, axis=0, keepdims=True)
    hn = (h - mu) * lax.rsqrt(var + 1e-5)
    f2 = jnp.maximum(gc_ref[...] * hn + bec_ref[...], 0.0)
    np1_ref[...] = lax.dot_general(
        f2, wl1_ref[...], (((1,), (1,)), ((), ())),
        precision=lax.Precision.HIGHEST) + bl1_ref[...]


_stage2a = pl.pallas_call(
    _stage2a_body,
    out_shape=jax.ShapeDtypeStruct((N, T), jnp.float32),
)


def _stage2b_body(feat_ref, np1_ref, wl0_ref, bl0_ref, batch_ref,
                  np_ref, wsi_ref):
    np0 = lax.dot_general(feat_ref[...], wl0_ref[...], (((1,), (1,)), ((), ())),
                          precision=lax.Precision.HIGHEST) + bl0_ref[...]
    np1 = np1_ref[...]
    np_ref[...] = np0 + np1

    mask = batch_ref[...] == lax.broadcasted_iota(jnp.int32, (1, G), 1)
    rows = []
    for t in range(T):
        m0 = jnp.max(jnp.where(mask, np0[:, t:t + 1], -jnp.inf), axis=0,
                     keepdims=True)
        m1 = jnp.max(jnp.where(mask, np1[:, t:t + 1], -jnp.inf), axis=0,
                     keepdims=True)
        rows.append(m0 + m1)
    wsi_ref[...] = jnp.concatenate(rows, axis=0)


_stage2b = pl.pallas_call(
    _stage2b_body,
    out_shape=[
        jax.ShapeDtypeStruct((N, T), jnp.float32),
        jax.ShapeDtypeStruct((T, G), jnp.float32),
    ],
)


def kernel(x, W1, b1, g1, be1, Wl0, bl0, Wc, bc, gc, bec, Wl1, bl1,
           edge_index, batch):
    src = edge_index[0]
    dst = edge_index[1]
    feature = _stage1(x, W1, b1.reshape(1, H), g1.reshape(1, H),
                      be1.reshape(1, H))
    zeros = jnp.zeros((NPAD, D), jnp.float32)
    agg2 = _sc_agg(src, dst, feature, zeros)
    np1 = _stage2a(feature, agg2, Wc, bc.reshape(1, H), gc.reshape(1, H),
                   bec.reshape(1, H), Wl1, bl1.reshape(1, T))
    node_pred, wsi_t = _stage2b(feature, np1, Wl0, bl0.reshape(1, T),
                                batch.reshape(N, 1))
    return (wsi_t.T, node_pred)
